# Initial kernel scaffold; baseline (speedup 1.0000x reference)
import jax
import jax.numpy as jnp
from jax.experimental import pallas as pl


def kernel(adj_edge_index, inputs, W):
    raise NotImplementedError("write your pallas kernel here")


# speedup vs baseline: 5.0975x; 5.0975x over previous
import functools

import jax
import jax.numpy as jnp
from jax import lax
from jax.experimental import pallas as pl
from jax.experimental.pallas import tpu as pltpu
from jax.experimental.pallas import tpu_sc as plsc

N = 10000
N_PAD = 10240
E = 320000
D_IN = 128
D_OUT = 128
DH = D_IN // 2

NC = 2
NS = 16
EDGES_PER_TILE = E // NS
K = 80
NB = EDGES_PER_TILE // K
ROWS_PER_TILE = N_PAD // NS
ZROWS = 128


def _sc_body(row_hbm, col_hbm, x_hbm, out_hbm, row_v, col_v, rows_v, zbuf, acc, sem):
    c = lax.axis_index("c")
    s = lax.axis_index("s")

    zeros = jnp.zeros((16,), jnp.float32)

    def _zero(i, carry):
        for j in range(DH // 16):
            zbuf[i, pl.ds(j * 16, 16)] = zeros
        return carry

    lax.fori_loop(0, ZROWS, _zero, 0)
    for kk in range(ROWS_PER_TILE // ZROWS):
        pltpu.sync_copy(zbuf, acc.at[pl.ds(s * ROWS_PER_TILE + kk * ZROWS, ZROWS)])
    plsc.subcore_barrier()

    pltpu.sync_copy(row_hbm.at[s], row_v)
    pltpu.sync_copy(col_hbm.at[s], col_v)

    def _edge_batch(j, carry):
        pltpu.async_copy(x_hbm.at[c].at[col_v.at[j]], rows_v, sem).wait()
        pltpu.sync_copy(rows_v, acc.at[row_v.at[j]], add=True)
        return carry

    lax.fori_loop(0, NB, _edge_batch, 0)
    plsc.subcore_barrier()

    for kk in range(ROWS_PER_TILE // ZROWS):
        sl = pl.ds(s * ROWS_PER_TILE + kk * ZROWS, ZROWS)
        pltpu.sync_copy(acc.at[sl], zbuf)
        pltpu.sync_copy(zbuf, out_hbm.at[c, sl])


_sc_segment_sum = functools.partial(
    pl.kernel,
    out_type=jax.ShapeDtypeStruct((NC, N_PAD, DH), jnp.float32),
    mesh=plsc.VectorSubcoreMesh(core_axis_name="c", subcore_axis_name="s"),
    compiler_params=pltpu.CompilerParams(use_tc_tiling_on_sc=False),
    scratch_types=[
        pltpu.VMEM((NB, K), jnp.int32),
        pltpu.VMEM((NB, K), jnp.int32),
        pltpu.VMEM((K, DH), jnp.float32),
        pltpu.VMEM((ZROWS, DH), jnp.float32),
        pltpu.VMEM_SHARED((N_PAD, DH), jnp.float32),
        pltpu.SemaphoreType.DMA,
    ],
)(_sc_body)


def _mm_body(p0_ref, p1_ref, w0_ref, w1_ref, o_ref):
    acc = jnp.dot(p0_ref[...], w0_ref[...], preferred_element_type=jnp.float32)
    acc += jnp.dot(p1_ref[...], w1_ref[...], preferred_element_type=jnp.float32)
    o_ref[...] = jnp.maximum(acc, 0.0)


_MM_BLOCK = 1024


def _mm_relu(p0, p1, w0, w1):
    return pl.pallas_call(
        _mm_body,
        grid=(N_PAD // _MM_BLOCK,),
        in_specs=[
            pl.BlockSpec((_MM_BLOCK, DH), lambda i: (i, 0)),
            pl.BlockSpec((_MM_BLOCK, DH), lambda i: (i, 0)),
            pl.BlockSpec((DH, D_OUT), lambda i: (0, 0)),
            pl.BlockSpec((DH, D_OUT), lambda i: (0, 0)),
        ],
        out_specs=pl.BlockSpec((_MM_BLOCK, D_OUT), lambda i: (i, 0)),
        out_shape=jax.ShapeDtypeStruct((N_PAD, D_OUT), jnp.float32),
    )(p0, p1, w0, w1)


def kernel(adj_edge_index, inputs, W):
    row = adj_edge_index[0].reshape(NS, NB, K)
    col = adj_edge_index[1].reshape(NS, NB, K)
    x2 = jnp.stack([inputs[:, :DH], inputs[:, DH:]], axis=0)
    partials = _sc_segment_sum(row, col, x2)
    return _mm_relu(partials[0], partials[1], W[:DH], W[DH:])[:N]


# speedup vs baseline: 6.2183x; 1.2199x over previous
import functools

import jax
import jax.numpy as jnp
from jax import lax
from jax.experimental import pallas as pl
from jax.experimental.pallas import tpu as pltpu
from jax.experimental.pallas import tpu_sc as plsc

N = 10000
N_PAD = 10240
E = 320000
D_IN = 128
D_OUT = 128
DH = D_IN // 2

NC = 2
NS = 16
EDGES_PER_TILE = E // NS
K = 80
NB = EDGES_PER_TILE // K
ROWS_PER_TILE = N_PAD // NS
ZROWS = 128


def _sc_body(row_hbm, col_hbm, x_hbm, out_hbm,
             row_v, col_v, rows0, rows1, zbuf,
             acc, sg0, sg1, ss0, ss1):
    c = lax.axis_index("c")
    s = lax.axis_index("s")

    zeros = jnp.zeros((16,), jnp.float32)

    def _zero(i, carry):
        for j in range(DH // 16):
            zbuf[i, pl.ds(j * 16, 16)] = zeros
        return carry

    lax.fori_loop(0, ZROWS, _zero, 0)
    for kk in range(ROWS_PER_TILE // ZROWS):
        pltpu.sync_copy(zbuf, acc.at[pl.ds(s * ROWS_PER_TILE + kk * ZROWS, ZROWS)])
    plsc.subcore_barrier()

    pltpu.sync_copy(row_hbm.at[s], row_v)
    pltpu.sync_copy(col_hbm.at[s], col_v)

    rows = (rows0, rows1)
    sg = (sg0, sg1)
    ss = (ss0, ss1)

    def _gather(j, b, sem):
        pltpu.async_copy(x_hbm.at[c].at[col_v.at[j]], rows[b], sem)

    def _gather_wait(b, sem):
        pltpu.make_async_copy(x_hbm.at[c].at[col_v.at[0]], rows[b], sem).wait()

    def _scatter(j, b, sem):
        pltpu.async_copy(rows[b], acc.at[row_v.at[j]], sem, add=True)

    def _scatter_wait(b, sem):
        pltpu.make_async_copy(x_hbm.at[c].at[col_v.at[0]], rows[b], sem).wait()

    _gather(0, 0, sg0)

    def _group(g, carry):
        for b in range(2):
            j = 2 * g + b
            _gather_wait(b, sg[b])
            _scatter(j, b, ss[b])
            nb = 1 - b
            @pl.when(jnp.logical_or(g > 0, b > 0))
            def _():
                _scatter_wait(nb, ss[nb])
            @pl.when(j + 1 < NB)
            def _():
                _gather(j + 1, nb, sg[nb])
        return carry

    lax.fori_loop(0, NB // 2, _group, 0)
    _scatter_wait(1, ss1)
    plsc.subcore_barrier()

    for kk in range(ROWS_PER_TILE // ZROWS):
        sl = pl.ds(s * ROWS_PER_TILE + kk * ZROWS, ZROWS)
        pltpu.sync_copy(acc.at[sl], zbuf)
        pltpu.sync_copy(zbuf, out_hbm.at[c, sl])


_sc_segment_sum = functools.partial(
    pl.kernel,
    out_type=jax.ShapeDtypeStruct((NC, N_PAD, DH), jnp.float32),
    mesh=plsc.VectorSubcoreMesh(core_axis_name="c", subcore_axis_name="s"),
    compiler_params=pltpu.CompilerParams(use_tc_tiling_on_sc=False),
    scratch_types=[
        pltpu.VMEM((NB, K), jnp.int32),
        pltpu.VMEM((NB, K), jnp.int32),
        pltpu.VMEM((K, DH), jnp.float32),
        pltpu.VMEM((K, DH), jnp.float32),
        pltpu.VMEM((ZROWS, DH), jnp.float32),
        pltpu.VMEM_SHARED((N_PAD, DH), jnp.float32),
        pltpu.SemaphoreType.DMA,
        pltpu.SemaphoreType.DMA,
        pltpu.SemaphoreType.DMA,
        pltpu.SemaphoreType.DMA,
    ],
)(_sc_body)


def _mm_body(p0_ref, p1_ref, w0_ref, w1_ref, o_ref):
    acc = jnp.dot(p0_ref[...], w0_ref[...], preferred_element_type=jnp.float32)
    acc += jnp.dot(p1_ref[...], w1_ref[...], preferred_element_type=jnp.float32)
    o_ref[...] = jnp.maximum(acc, 0.0)


_MM_BLOCK = 1000


def _mm_relu(p0, p1, w0, w1):
    return pl.pallas_call(
        _mm_body,
        grid=(N // _MM_BLOCK,),
        in_specs=[
            pl.BlockSpec((_MM_BLOCK, DH), lambda i: (i, 0)),
            pl.BlockSpec((_MM_BLOCK, DH), lambda i: (i, 0)),
            pl.BlockSpec((DH, D_OUT), lambda i: (0, 0)),
            pl.BlockSpec((DH, D_OUT), lambda i: (0, 0)),
        ],
        out_specs=pl.BlockSpec((_MM_BLOCK, D_OUT), lambda i: (i, 0)),
        out_shape=jax.ShapeDtypeStruct((N, D_OUT), jnp.float32),
    )(p0, p1, w0, w1)


def kernel(adj_edge_index, inputs, W):
    row = adj_edge_index[0].reshape(NS, NB, K)
    col = adj_edge_index[1].reshape(NS, NB, K)
    x2 = jnp.stack([inputs[:, :DH], inputs[:, DH:]], axis=0)
    partials = _sc_segment_sum(row, col, x2)
    return _mm_relu(partials[0], partials[1], W[:DH], W[DH:])


# speedup vs baseline: 12.5597x; 2.0198x over previous
import functools

import jax
import jax.numpy as jnp
from jax import lax
from jax.experimental import pallas as pl
from jax.experimental.pallas import tpu as pltpu
from jax.experimental.pallas import tpu_sc as plsc

N = 10000
N_PAD = 10240
E = 320000
D_IN = 128
D_OUT = 128
DH = D_IN // 2

NC = 2
NS = 16
K = 128
NB = 160
EPT = NB * K
EPT_REAL = E // NS
PAD = EPT - EPT_REAL
NBUF = 4
ROWS_PER_TILE = N_PAD // NS
ZROWS = 128


def _sc_body(row_hbm, col_hbm, x_hbm, out_hbm,
             row_v, col_v, rows0, rows1, rows2, rows3, zbuf,
             acc, sg0, sg1, sg2, sg3, ss0, ss1, ss2, ss3):
    c = lax.axis_index("c")
    s = lax.axis_index("s")

    zeros = jnp.zeros((16,), jnp.float32)

    def _zero(i, carry):
        for j in range(DH // 16):
            zbuf[i, pl.ds(j * 16, 16)] = zeros
        return carry

    lax.fori_loop(0, ZROWS, _zero, 0)
    for kk in range(ROWS_PER_TILE // ZROWS):
        pltpu.sync_copy(zbuf, acc.at[pl.ds(s * ROWS_PER_TILE + kk * ZROWS, ZROWS)])
    plsc.subcore_barrier()

    pltpu.sync_copy(row_hbm.at[s], row_v)
    pltpu.sync_copy(col_hbm.at[c, s], col_v)

    rows = (rows0, rows1, rows2, rows3)
    sg = (sg0, sg1, sg2, sg3)
    ss = (ss0, ss1, ss2, ss3)

    def _gather(j, b):
        pltpu.async_copy(x_hbm.at[col_v.at[j]], rows[b], sg[b])

    def _gather_wait(b):
        pltpu.make_async_copy(x_hbm.at[col_v.at[0]], rows[b], sg[b]).wait()

    def _scatter(j, b):
        pltpu.async_copy(rows[b], acc.at[row_v.at[j]], ss[b], add=True)

    def _scatter_wait(b):
        pltpu.make_async_copy(x_hbm.at[col_v.at[0]], rows[b], ss[b]).wait()

    _gather(0, 0)
    _gather(1, 1)

    def _group(g, carry):
        for b in range(NBUF):
            j = NBUF * g + b
            nb = (b + 2) % NBUF
            if b < 2:
                @pl.when(g > 0)
                def _():
                    _scatter_wait(nb)
                _gather(j + 2, nb)
            else:
                _scatter_wait(nb)
                @pl.when(g < NB // NBUF - 1)
                def _():
                    _gather(j + 2, nb)
            _gather_wait(b)
            _scatter(j, b)
        return carry

    lax.fori_loop(0, NB // NBUF, _group, 0)
    _scatter_wait(2)
    _scatter_wait(3)
    plsc.subcore_barrier()

    for kk in range(ROWS_PER_TILE // ZROWS):
        sl = pl.ds(s * ROWS_PER_TILE + kk * ZROWS, ZROWS)
        pltpu.sync_copy(acc.at[sl], zbuf)
        pltpu.sync_copy(zbuf, out_hbm.at[sl, pl.ds(c * DH, DH)])


_sc_segment_sum = functools.partial(
    pl.kernel,
    out_type=jax.ShapeDtypeStruct((N_PAD, D_IN), jnp.float32),
    mesh=plsc.VectorSubcoreMesh(core_axis_name="c", subcore_axis_name="s"),
    compiler_params=pltpu.CompilerParams(use_tc_tiling_on_sc=False),
    scratch_types=[
        pltpu.VMEM((NB, K), jnp.int32),
        pltpu.VMEM((NB, K), jnp.int32),
        pltpu.VMEM((K, DH), jnp.float32),
        pltpu.VMEM((K, DH), jnp.float32),
        pltpu.VMEM((K, DH), jnp.float32),
        pltpu.VMEM((K, DH), jnp.float32),
        pltpu.VMEM((ZROWS, DH), jnp.float32),
        pltpu.VMEM_SHARED((N_PAD, DH), jnp.float32),
        pltpu.SemaphoreType.DMA,
        pltpu.SemaphoreType.DMA,
        pltpu.SemaphoreType.DMA,
        pltpu.SemaphoreType.DMA,
        pltpu.SemaphoreType.DMA,
        pltpu.SemaphoreType.DMA,
        pltpu.SemaphoreType.DMA,
        pltpu.SemaphoreType.DMA,
    ],
)(_sc_body)


def _mm_body(p_ref, w_ref, o_ref):
    o_ref[...] = jnp.maximum(
        jnp.dot(p_ref[...], w_ref[...], preferred_element_type=jnp.float32), 0.0
    )


_MM_BLOCK = 1000


def _mm_relu(p, w):
    return pl.pallas_call(
        _mm_body,
        grid=(N // _MM_BLOCK,),
        in_specs=[
            pl.BlockSpec((_MM_BLOCK, D_IN), lambda i: (i, 0)),
            pl.BlockSpec((D_IN, D_OUT), lambda i: (0, 0)),
        ],
        out_specs=pl.BlockSpec((_MM_BLOCK, D_OUT), lambda i: (i, 0)),
        out_shape=jax.ShapeDtypeStruct((N, D_OUT), jnp.float32),
    )(p, w)


def kernel(adj_edge_index, inputs, W):
    row_t = adj_edge_index[0].reshape(NS, EPT_REAL)
    col_t = adj_edge_index[1].reshape(NS, EPT_REAL)
    pad_r = N + (jnp.arange(PAD, dtype=jnp.int32) % (N_PAD - N))
    pad_c = (jnp.arange(PAD, dtype=jnp.int32) * 41) % N
    row_p = jnp.concatenate(
        [row_t, jnp.broadcast_to(pad_r, (NS, PAD))], axis=1).reshape(NS, NB, K)
    col_p = jnp.concatenate(
        [col_t, jnp.broadcast_to(pad_c, (NS, PAD))], axis=1).reshape(NS, NB, K)
    col2 = jnp.stack([col_p * 2, col_p * 2 + 1])
    x_r = inputs.reshape(2 * N, DH)
    agg = _sc_segment_sum(row_p, col2, x_r)
    return _mm_relu(agg, W)
